# Initial kernel scaffold; baseline (speedup 1.0000x reference)
import jax
import jax.numpy as jnp
from jax.experimental import pallas as pl


def kernel(dxEmb, classEmb, anEmb, reEmb, Wa, ba, Wcomb, bcomb, Wcls, bcls, leaves, ancestors, relations, permute_index):
    raise NotImplementedError("write your pallas kernel here")


# speedup vs baseline: 4.7840x; 4.7840x over previous
import jax
import jax.numpy as jnp
from jax import lax
from jax.experimental import pallas as pl
from jax.experimental.pallas import tpu as pltpu
from jax.experimental.pallas import tpu_sc as plsc

NC = 2
NS = 16
NW = NC * NS
L = 16
D = 128
H = 80
NH = 4
C = NH * H
VP = NW * C
NJ = D // 16
V0 = 10000
R32 = 32
BLK = 1024


def _mesh():
    return plsc.VectorSubcoreMesh(
        core_axis_name="c", subcore_axis_name="s",
        num_cores=NC, num_subcores=NS)


def _pre_body(x_ref, w_ref, b_ref, r_ref, re_ref, cv_ref, rs_ref):
    x = x_ref[...]
    cv_ref[0, 0, :] = jnp.sum(x * w_ref[...][None, :], axis=1) + b_ref[0]
    rel = r_ref[...]
    iota_r = lax.broadcasted_iota(jnp.int32, (1, R32), 1)
    counts = jnp.zeros((BLK, R32), jnp.float32)
    for l in range(L):
        counts += (rel[:, l][:, None] == iota_r).astype(jnp.float32)
    rs_ref[...] = jnp.dot(counts, re_ref[...],
                          preferred_element_type=jnp.float32)


def _pre(classEmbP, wcls, b1, relP, reEmbP):
    cvec, relsum = pl.pallas_call(
        _pre_body,
        out_shape=[
            jax.ShapeDtypeStruct((VP // BLK, 1, BLK), jnp.float32),
            jax.ShapeDtypeStruct((VP, D), jnp.float32),
        ],
        grid=(VP // BLK,),
        in_specs=[
            pl.BlockSpec((BLK, D), lambda i: (i, 0)),
            pl.BlockSpec((D,), lambda i: (0,)),
            pl.BlockSpec(memory_space=pltpu.SMEM),
            pl.BlockSpec((BLK, L), lambda i: (i, 0)),
            pl.BlockSpec((R32, D), lambda i: (0, 0)),
        ],
        out_specs=[
            pl.BlockSpec((1, 1, BLK), lambda i: (i, 0, 0)),
            pl.BlockSpec((BLK, D), lambda i: (i, 0)),
        ],
    )(classEmbP, wcls, b1, relP, reEmbP)
    return cvec.reshape(VP), relsum


def _k1_body(anEmb, cvec, relsum, anc_t, lv2, iota2,
             temp_out,
             anc_v, lv_v, iota_v, idx_v, cvec_v,
             acc, gbufa, gbufb, abuf, rbuf, sema):
    sid = lax.axis_index("s")
    wid = sid * NC + lax.axis_index("c")
    base = wid * C
    bufs = [gbufa, gbufb]

    pltpu.sync_copy(anc_t.at[:, pl.ds(wid * NH, NH), :], anc_v)
    pltpu.sync_copy(lv2.at[pl.ds(wid * NH, NH)], lv_v)
    pltpu.sync_copy(iota2, iota_v)
    pltpu.sync_copy(cvec, cvec_v)

    off = sid * C
    for h in range(NH):
        for t in range(H // 16):
            idx_v[h, pl.ds(t * 16, 16)] = iota_v[h, pl.ds(t * 16, 16)] + off

    for h in range(NH):
        pltpu.async_copy(anEmb.at[anc_v.at[0, h]], gbufb, sema).wait()
        pltpu.sync_copy(gbufb, acc.at[idx_v.at[h]])

    pltpu.async_copy(anEmb.at[anc_v.at[1, 0]], gbufa, sema)

    def l_body(l, carry):
        for h in range(NH):
            cur = bufs[h % 2]
            nk = l * NH + h + 1

            @pl.when(nk < L * NH)
            def _issue():
                pltpu.async_copy(anEmb.at[anc_v.at[nk // NH, nk % NH]],
                                 bufs[(h + 1) % 2], sema)

            pltpu.make_async_copy(anEmb.at[pl.ds(0, H)], cur, sema).wait()
            pltpu.sync_copy(cur, acc.at[idx_v.at[h]], add=True)
        return carry
    lax.fori_loop(1, L, l_body, 0)

    for h in range(NH):
        pltpu.sync_copy(acc.at[pl.ds(off + h * H, H)], abuf)
        pltpu.sync_copy(relsum.at[pl.ds(base + h * H, H)], rbuf)

        def t_body(t, carry, h=h):
            coe16 = plsc.load_gather(cvec_v, [lv_v[h, pl.ds(t * 16, 16)]])
            for n in range(16):
                m = t * 16 + n
                coe = coe16[n]
                for j in range(NJ):
                    abuf[m, pl.ds(j * 16, 16)] = (
                        abuf[m, pl.ds(j * 16, 16)]
                        + coe * rbuf[m, pl.ds(j * 16, 16)])
            return carry
        lax.fori_loop(0, H // 16, t_body, 0)

        zh, zn = divmod(V0 - (NW - 1) * C, H)
        if h == zh:
            @pl.when(wid == NW - 1)
            def _zero_row():
                for j in range(NJ):
                    abuf[zn, pl.ds(j * 16, 16)] = jnp.zeros((16,), jnp.float32)

        pltpu.sync_copy(abuf, temp_out.at[pl.ds(base + h * H, H)])


def _k2_body(temp_hbm, perm2, out_hbm, idx_v, rows_v, sem):
    wid = lax.axis_index("s") * NC + lax.axis_index("c")
    base = wid * C
    pltpu.sync_copy(perm2.at[pl.ds(wid * NH, NH)], idx_v)
    for h in range(NH):
        pltpu.async_copy(temp_hbm.at[idx_v.at[h]],
                         rows_v.at[pl.ds(h * H, H)], sem).wait()
    pltpu.sync_copy(rows_v, out_hbm.at[pl.ds(base, C)])


@jax.jit
def _run(anEmb, reEmbP, classEmbP, wcls, b1, anc_t, relP, lv2, iota2, perm2):
    cvec, relsum = _pre(classEmbP, wcls, b1, relP, reEmbP)

    k1 = pl.kernel(
        _k1_body,
        out_type=jax.ShapeDtypeStruct((VP, D), jnp.float32),
        mesh=_mesh(),
        compiler_params=pltpu.CompilerParams(needs_layout_passes=False),
        scratch_types=[
            pltpu.VMEM((L, NH, H), jnp.int32),
            pltpu.VMEM((NH, H), jnp.int32),
            pltpu.VMEM((NH, H), jnp.int32),
            pltpu.VMEM((NH, H), jnp.int32),
            pltpu.VMEM((VP,), jnp.float32),
            pltpu.VMEM_SHARED((NS * C, D), jnp.float32),
            pltpu.VMEM((H, D), jnp.float32),
            pltpu.VMEM((H, D), jnp.float32),
            pltpu.VMEM((H, D), jnp.float32),
            pltpu.VMEM((H, D), jnp.float32),
            pltpu.SemaphoreType.DMA,
        ],
    )
    temp = k1(anEmb, cvec, relsum, anc_t, lv2, iota2)

    k2 = pl.kernel(
        _k2_body,
        out_type=jax.ShapeDtypeStruct((VP, D), jnp.float32),
        mesh=_mesh(),
        compiler_params=pltpu.CompilerParams(needs_layout_passes=False),
        scratch_types=[
            pltpu.VMEM((NH, H), jnp.int32),
            pltpu.VMEM((C, D), jnp.float32),
            pltpu.SemaphoreType.DMA,
        ],
    )
    return k2(temp, perm2)


def kernel(dxEmb, classEmb, anEmb, reEmb, Wa, ba, Wcomb, bcomb, Wcls, bcls,
           leaves, ancestors, relations, permute_index):
    V = classEmb.shape[0]
    pad = VP - V

    anc_t = jnp.pad(ancestors.astype(jnp.int32), ((0, pad), (0, 0))).T
    anc_t = anc_t.reshape(L, NW * NH, H)
    relP = jnp.pad(relations.astype(jnp.int32), ((0, pad), (0, 0)))
    lv2 = jnp.pad(leaves[:, 0].astype(jnp.int32), (0, pad))
    lv2 = lv2.reshape(NW * NH, H)
    perm2 = jnp.pad(permute_index.astype(jnp.int32), (0, VP - (V + 1)),
                    constant_values=V).reshape(NW * NH, H)
    iota2 = jnp.arange(C, dtype=jnp.int32).reshape(NH, H)
    classEmbP = jnp.pad(classEmb.astype(jnp.float32), ((0, pad), (0, 0)))
    reEmbP = jnp.pad(reEmb.astype(jnp.float32),
                     ((0, R32 - reEmb.shape[0]), (0, 0)))
    wcls = Wcls[0].astype(jnp.float32)
    b1 = bcls.astype(jnp.float32)

    out = _run(anEmb.astype(jnp.float32), reEmbP, classEmbP, wcls, b1,
               anc_t, relP, lv2, iota2, perm2)
    return out[:V + 1]


# speedup vs baseline: 5.5038x; 1.1505x over previous
import jax
import jax.numpy as jnp
from jax import lax
from jax.experimental import pallas as pl
from jax.experimental.pallas import tpu as pltpu
from jax.experimental.pallas import tpu_sc as plsc

NC = 2
NS = 16
NW = NC * NS
L = 16
D = 128
H = 80
NH = 4
C = NH * H
VP = NW * C
NJ = D // 16
V0 = 10000
R32 = 32
BLK = 1024
K = L * NH


def _mesh():
    return plsc.VectorSubcoreMesh(
        core_axis_name="c", subcore_axis_name="s",
        num_cores=NC, num_subcores=NS)


def _pre_body(x_ref, w_ref, b_ref, r_ref, re_ref, cv_ref, rs_ref):
    i = pl.program_id(0)
    x = x_ref[...]
    cv = jnp.sum(x * w_ref[...][None, :], axis=1) + b_ref[0]
    rows = i * BLK + lax.broadcasted_iota(jnp.int32, (BLK,), 0)
    cv_ref[0, 0, :] = jnp.where(rows >= V0, 0.0, cv)
    rel = r_ref[...]
    iota_r = lax.broadcasted_iota(jnp.int32, (1, R32), 1)
    counts = jnp.zeros((BLK, R32), jnp.float32)
    for l in range(L):
        counts += (rel[:, l][:, None] == iota_r).astype(jnp.float32)
    rs_ref[...] = jnp.dot(counts, re_ref[...],
                          preferred_element_type=jnp.float32)


def _pre(classEmbP, wcls, b1, relP, reEmbP):
    cvec, relsum2 = pl.pallas_call(
        _pre_body,
        out_shape=[
            jax.ShapeDtypeStruct((VP // BLK, 1, BLK), jnp.float32),
            jax.ShapeDtypeStruct((2 * VP, D), jnp.float32),
        ],
        grid=(VP // BLK,),
        in_specs=[
            pl.BlockSpec((BLK, D), lambda i: (i, 0)),
            pl.BlockSpec((D,), lambda i: (0,)),
            pl.BlockSpec(memory_space=pltpu.SMEM),
            pl.BlockSpec((BLK, L), lambda i: (i, 0)),
            pl.BlockSpec((R32, D), lambda i: (0, 0)),
        ],
        out_specs=[
            pl.BlockSpec((1, 1, BLK), lambda i: (i, 0, 0)),
            pl.BlockSpec((BLK, D), lambda i: (i, 0)),
        ],
    )(classEmbP, wcls, b1, relP, reEmbP)
    return cvec.reshape(VP), relsum2


def _k1_body(anEmb, anc_t, iota2,
             acc_out,
             anc_v, iota_v, idx_v,
             acc, g0, g1, g2, g3, semg, sems):
    sid = lax.axis_index("s")
    wid = sid * NC + lax.axis_index("c")
    base = wid * C
    gbufs = [g0, g1, g2, g3]

    pltpu.sync_copy(anc_t.at[:, pl.ds(wid * NH, NH), :], anc_v)
    pltpu.sync_copy(iota2, iota_v)

    for k in range(2):
        pltpu.async_copy(anEmb.at[anc_v.at[0, k]], gbufs[k], semg)

    off = sid * C
    for h in range(NH):
        for t in range(H // 16):
            idx_v[h, pl.ds(t * 16, 16)] = iota_v[h, pl.ds(t * 16, 16)] + off

    def z_body(n, carry):
        for j in range(NJ):
            g3[n, pl.ds(j * 16, 16)] = jnp.zeros((16,), jnp.float32)
        return carry
    lax.fori_loop(0, H, z_body, 0)
    for h in range(NH):
        pltpu.sync_copy(g3, acc.at[pl.ds(off + h * H, H)])

    def m_body(m, carry):
        for i in range(NH):
            k = m * NH + i
            b = gbufs[i]

            @pl.when(k >= 2)
            def _drain_one_scatter():
                pltpu.make_async_copy(b, acc.at[pl.ds(0, H)], sems).wait()

            pltpu.make_async_copy(anEmb.at[pl.ds(0, H)], b, semg).wait()
            pltpu.async_copy(b, acc.at[idx_v.at[i]], sems, add=True)

            @pl.when(k + 2 < K)
            def _issue_next_gather():
                pltpu.async_copy(
                    anEmb.at[anc_v.at[m + (i + 2) // NH, (i + 2) % NH]],
                    gbufs[(i + 2) % NH], semg)
        return carry
    lax.fori_loop(0, K // NH, m_body, 0)

    for _ in range(2):
        pltpu.make_async_copy(g0, acc.at[pl.ds(0, H)], sems).wait()

    zr = V0 - (NW - 1) * C

    @pl.when(wid == NW - 1)
    def _zero_row():
        for j in range(NJ):
            g3[0, pl.ds(j * 16, 16)] = jnp.zeros((16,), jnp.float32)
        pltpu.sync_copy(g3.at[pl.ds(0, 1)], acc.at[pl.ds(off + zr, 1)])

    pltpu.sync_copy(acc.at[pl.ds(off, C)], acc_out.at[pl.ds(base, C)])


def _k2_body(ancsum, relsum2, cvec, lv1, perm2, out_hbm,
             idx_v, cvec_v, lv_v, abuf, rbuf, sema, semb):
    wid = lax.axis_index("s") * NC + lax.axis_index("c")
    base = wid * C
    pltpu.sync_copy(perm2.at[pl.ds(wid * NH, NH)], idx_v)
    pltpu.sync_copy(cvec, cvec_v)
    pltpu.sync_copy(lv1, lv_v)

    for h in range(NH):
        da = pltpu.async_copy(ancsum.at[idx_v.at[h]], abuf, sema)
        db = pltpu.async_copy(relsum2.at[idx_v.at[h]], rbuf, semb)
        da.wait()
        db.wait()

        def t_body(t, carry, h=h):
            p16 = idx_v[h, pl.ds(t * 16, 16)]
            lv16 = plsc.load_gather(lv_v, [p16])
            coe16 = plsc.load_gather(cvec_v, [lv16])
            for n in range(16):
                m = t * 16 + n
                coe = coe16[n]
                for j in range(NJ):
                    abuf[m, pl.ds(j * 16, 16)] = (
                        abuf[m, pl.ds(j * 16, 16)]
                        + coe * rbuf[m, pl.ds(j * 16, 16)])
            return carry
        lax.fori_loop(0, H // 16, t_body, 0)

        pltpu.sync_copy(abuf, out_hbm.at[pl.ds(base + h * H, H)])


@jax.jit
def _run(anEmb, reEmbP, classEmbP, wcls, b1, anc_t, relP, lv1, iota2, perm2):
    cvec, relsum2 = _pre(classEmbP, wcls, b1, relP, reEmbP)

    k1 = pl.kernel(
        _k1_body,
        out_type=jax.ShapeDtypeStruct((VP, D), jnp.float32),
        mesh=_mesh(),
        compiler_params=pltpu.CompilerParams(needs_layout_passes=False),
        scratch_types=[
            pltpu.VMEM((L, NH, H), jnp.int32),
            pltpu.VMEM((NH, H), jnp.int32),
            pltpu.VMEM((NH, H), jnp.int32),
            pltpu.VMEM_SHARED((NS * C, D), jnp.float32),
        ] + [pltpu.VMEM((H, D), jnp.float32) for _ in range(4)] + [
            pltpu.SemaphoreType.DMA,
            pltpu.SemaphoreType.DMA,
        ],
    )
    ancsum = k1(anEmb, anc_t, iota2)

    k2 = pl.kernel(
        _k2_body,
        out_type=jax.ShapeDtypeStruct((VP, D), jnp.float32),
        mesh=_mesh(),
        compiler_params=pltpu.CompilerParams(needs_layout_passes=False),
        scratch_types=[
            pltpu.VMEM((NH, H), jnp.int32),
            pltpu.VMEM((VP,), jnp.float32),
            pltpu.VMEM((VP,), jnp.int32),
            pltpu.VMEM((H, D), jnp.float32),
            pltpu.VMEM((H, D), jnp.float32),
            pltpu.SemaphoreType.DMA,
            pltpu.SemaphoreType.DMA,
        ],
    )
    return k2(ancsum, relsum2, cvec, lv1, perm2)


def kernel(dxEmb, classEmb, anEmb, reEmb, Wa, ba, Wcomb, bcomb, Wcls, bcls,
           leaves, ancestors, relations, permute_index):
    V = classEmb.shape[0]
    pad = VP - V

    anc_t = jnp.pad(ancestors.astype(jnp.int32), ((0, pad), (0, 0))).T
    anc_t = jnp.pad(anc_t.reshape(L, NW * NH, H), ((0, 0), (0, 1536), (0, 0)))
    relP = jnp.pad(relations.astype(jnp.int32), ((0, pad), (0, 0)))
    lv1 = jnp.pad(leaves[:, 0].astype(jnp.int32), (0, pad))
    perm2 = jnp.pad(permute_index.astype(jnp.int32), (0, VP - (V + 1)),
                    constant_values=V).reshape(NW * NH, H)
    iota2 = jnp.arange(C, dtype=jnp.int32).reshape(NH, H)
    classEmbP = jnp.pad(classEmb.astype(jnp.float32), ((0, pad), (0, 0)))
    reEmbP = jnp.pad(reEmb.astype(jnp.float32),
                     ((0, R32 - reEmb.shape[0]), (0, 0)))
    wcls = Wcls[0].astype(jnp.float32)
    b1 = bcls.astype(jnp.float32)

    out = _run(anEmb.astype(jnp.float32), reEmbP, classEmbP, wcls, b1,
               anc_t, relP, lv1, iota2, perm2)
    return out[:V + 1]
